# Initial kernel scaffold; baseline (speedup 1.0000x reference)
import jax
import jax.numpy as jnp
from jax.experimental import pallas as pl


def kernel(x, edge_index, edge_attr, batch, bit_width, Wrel, brel, Wroot, Wd, bd, Wout, bout):
    raise NotImplementedError("write your pallas kernel here")


# speedup vs baseline: 3.5077x; 3.5077x over previous
import functools

import jax
import jax.numpy as jnp
from jax import lax
from jax.experimental import pallas as pl
from jax.experimental.pallas import tpu as pltpu
from jax.experimental.pallas import tpu_sc as plsc

N = 10000
E = 160000
NG = 64
NC, NS = 2, 16
W = NC * NS
EPW = E // W
KB = 128
NB_FULL = EPW // KB
TAIL = EPW - NB_FULL * KB
CP = 624
REM = N - NS * CP
ZR = 208
BN = 1000
f32 = jnp.float32


def _minmax(a):
    def body(a_ref, lo_ref, hi_ref):
        v = a_ref[...]
        lo_ref[...] = jnp.min(v).reshape(1, 1)
        hi_ref[...] = jnp.max(v).reshape(1, 1)
    return pl.pallas_call(
        body,
        out_shape=(jax.ShapeDtypeStruct((1, 1), f32),
                   jax.ShapeDtypeStruct((1, 1), f32)),
    )(a)


def _quantmm(x, lo, hi, qm, Wr, Wroot, br, pre, nchunk, dc):
    n, fin = x.shape
    fout = Wr.shape[0]
    dpad = nchunk * dc
    grid = (n // BN,)

    def body(lo_ref, hi_ref, qm_ref, x_ref, wr_ref, wrt_ref, br_ref,
             tab_ref, root_ref):
        lo_v = lo_ref[0, 0]
        hi_v = hi_ref[0, 0]
        qmax = qm_ref[0, 0]
        s = (hi_v - lo_v) / qmax
        zp = jnp.round(-lo_v / s)
        xb = x_ref[...]
        xq = (jnp.clip(jnp.round(xb / s) + zp, 0.0, qmax) - zp) * s
        root_ref[...] = lax.dot_general(
            xq, wrt_ref[...], (((1,), (1,)), ((), ())),
            preferred_element_type=f32) + br_ref[...]
        if pre:
            y = lax.dot_general(xq, wr_ref[...], (((1,), (1,)), ((), ())),
                                preferred_element_type=f32)
        else:
            y = xq
            if fin < dpad:
                y = jnp.concatenate(
                    [y, jnp.zeros((BN, dpad - fin), f32)], axis=1)
        for c in range(nchunk):
            tab_ref[c, :, :] = y[:, c * dc:(c + 1) * dc]

    return pl.pallas_call(
        body,
        grid=grid,
        in_specs=[
            pl.BlockSpec((1, 1), lambda i: (0, 0)),
            pl.BlockSpec((1, 1), lambda i: (0, 0)),
            pl.BlockSpec((1, 1), lambda i: (0, 0)),
            pl.BlockSpec((BN, fin), lambda i: (i, 0)),
            pl.BlockSpec((fout, fin), lambda i: (0, 0)),
            pl.BlockSpec((fout, fin), lambda i: (0, 0)),
            pl.BlockSpec((1, fout), lambda i: (0, 0)),
        ],
        out_specs=[
            pl.BlockSpec((nchunk, BN, dc), lambda i: (0, i, 0)),
            pl.BlockSpec((BN, fout), lambda i: (i, 0)),
        ],
        out_shape=[
            jax.ShapeDtypeStruct((nchunk, n, dc), f32),
            jax.ShapeDtypeStruct((n, fout), f32),
        ],
    )(lo, hi, qm, x, Wr, Wroot, br)


def _eaquant(ea2d, lo, hi, qm):
    def body(lo_ref, hi_ref, qm_ref, ea_ref, out_ref, loe_ref, hie_ref):
        lo_v = lo_ref[0, 0]
        hi_v = hi_ref[0, 0]
        qmax = qm_ref[0, 0]
        s = (hi_v - lo_v) / qmax
        zp = jnp.round(-lo_v / s)
        e = ea_ref[...]
        eq = (jnp.clip(jnp.round(e / s) + zp, 0.0, qmax) - zp) * s
        out_ref[...] = eq
        loe_ref[...] = jnp.min(eq).reshape(1, 1)
        hie_ref[...] = jnp.max(eq).reshape(1, 1)
    return pl.pallas_call(
        body,
        out_shape=(jax.ShapeDtypeStruct(ea2d.shape, f32),
                   jax.ShapeDtypeStruct((1, 1), f32),
                   jax.ShapeDtypeStruct((1, 1), f32)),
    )(lo, hi, qm, ea2d)


def _combine(parts, root, Wr, fin, nchunk, dc):
    n, fout = root.shape
    grid = (n // BN,)
    post = Wr is not None

    def body(*refs):
        if post:
            p_ref, root_ref, wr_ref, z_ref, lo_ref, hi_ref = refs
        else:
            p_ref, root_ref, z_ref, lo_ref, hi_ref = refs
        p = p_ref[...]
        agg = p[0] + p[1]
        agg2 = jnp.concatenate([agg[c] for c in range(nchunk)], axis=1)
        if post:
            agg2 = agg2[:, :fin]
            zb = lax.dot_general(agg2, wr_ref[...], (((1,), (1,)), ((), ())),
                                 preferred_element_type=f32) + root_ref[...]
        else:
            zb = agg2 + root_ref[...]
        zb = jnp.maximum(zb, 0.0)
        z_ref[...] = zb
        mn = jnp.min(zb).reshape(1, 1)
        mx = jnp.max(zb).reshape(1, 1)
        i = pl.program_id(0)

        @pl.when(i == 0)
        def _():
            lo_ref[...] = mn
            hi_ref[...] = mx

        @pl.when(i > 0)
        def _():
            lo_ref[...] = jnp.minimum(lo_ref[...], mn)
            hi_ref[...] = jnp.maximum(hi_ref[...], mx)

    in_specs = [
        pl.BlockSpec((2, nchunk, BN, dc), lambda i: (0, 0, i, 0)),
        pl.BlockSpec((BN, fout), lambda i: (i, 0)),
    ]
    args = [parts, root]
    if post:
        in_specs.append(pl.BlockSpec((fout, fin), lambda i: (0, 0)))
        args.append(Wr)
    return pl.pallas_call(
        body,
        grid=grid,
        in_specs=in_specs,
        out_specs=[
            pl.BlockSpec((BN, fout), lambda i: (i, 0)),
            pl.BlockSpec((1, 1), lambda i: (0, 0)),
            pl.BlockSpec((1, 1), lambda i: (0, 0)),
        ],
        out_shape=[
            jax.ShapeDtypeStruct((n, fout), f32),
            jax.ShapeDtypeStruct((1, 1), f32),
            jax.ShapeDtypeStruct((1, 1), f32),
        ],
    )(*args)


def _pool(z, batch2d):
    n, fdim = z.shape
    grid = (n // BN,)

    def body(z_ref, b_ref, sums_ref, cnt_ref):
        zb = z_ref[...]
        bb = b_ref[...]
        g = lax.broadcasted_iota(jnp.int32, (BN, NG), 1)
        mask = (bb == g).astype(f32)
        s = lax.dot_general(mask, zb, (((0,), (0,)), ((), ())),
                            preferred_element_type=f32)
        c = lax.dot_general(mask, jnp.ones((BN, 1), f32),
                            (((0,), (0,)), ((), ())),
                            preferred_element_type=f32)
        i = pl.program_id(0)

        @pl.when(i == 0)
        def _():
            sums_ref[...] = s
            cnt_ref[...] = c

        @pl.when(i > 0)
        def _():
            sums_ref[...] += s
            cnt_ref[...] += c

    return pl.pallas_call(
        body,
        grid=grid,
        in_specs=[
            pl.BlockSpec((BN, fdim), lambda i: (i, 0)),
            pl.BlockSpec((BN, 1), lambda i: (i, 0)),
        ],
        out_specs=[
            pl.BlockSpec((NG, fdim), lambda i: (0, 0)),
            pl.BlockSpec((NG, 1), lambda i: (0, 0)),
        ],
        out_shape=[
            jax.ShapeDtypeStruct((NG, fdim), f32),
            jax.ShapeDtypeStruct((NG, 1), f32),
        ],
    )(z, batch2d)


def _mlp(sums, counts, loe, hie, qm, Wd, bd, Wout, bout):
    def body(sums_ref, cnt_ref, loe_ref, hie_ref, qm_ref,
             w0, w1, w2, b0, b1, b2, wo, bo, out_ref):
        x = sums_ref[...] / jnp.maximum(cnt_ref[...], 1.0)
        le = loe_ref[0, 0]
        he = hie_ref[0, 0]
        qmax = qm_ref[0, 0]
        ws = [w0, w1, w2]
        bs = [b0, b1, b2]
        for j in range(3):
            lo_v = jnp.minimum(jnp.min(x), le)
            hi_v = jnp.maximum(jnp.max(x), he)
            s = (hi_v - lo_v) / qmax
            zp = jnp.round(-lo_v / s)
            xq = (jnp.clip(jnp.round(x / s) + zp, 0.0, qmax) - zp) * s
            x = lax.dot_general(xq, ws[j][...], (((1,), (1,)), ((), ())),
                                preferred_element_type=f32) + bs[j][...]
            x = jnp.maximum(x, 0.0)
        lo_v = jnp.minimum(jnp.min(x), le)
        hi_v = jnp.maximum(jnp.max(x), he)
        s = (hi_v - lo_v) / qmax
        zp = jnp.round(-lo_v / s)
        xq = (jnp.clip(jnp.round(x / s) + zp, 0.0, qmax) - zp) * s
        out_ref[...] = (jnp.sum(xq * wo[...], axis=1, keepdims=True)
                        + bo[0, 0])

    return pl.pallas_call(
        body,
        out_shape=jax.ShapeDtypeStruct((NG, 1), f32),
    )(sums, counts, loe, hie, qm, Wd[0], Wd[1], Wd[2],
      bd[0], bd[1], bd[2], Wout, bout)


@functools.lru_cache(maxsize=None)
def _make_scatter(nchunk, dc):
    mesh = plsc.VectorSubcoreMesh(core_axis_name="c", subcore_axis_name="s")
    nv = dc // 16

    @functools.partial(
        pl.kernel,
        out_type=jax.ShapeDtypeStruct((NC, nchunk, N, dc), f32),
        mesh=mesh,
        compiler_params=pltpu.CompilerParams(use_tc_tiling_on_sc=False),
        scratch_types=[
            pltpu.VMEM_SHARED((N, dc), f32),
            pltpu.VMEM((KB,), jnp.int32),
            pltpu.VMEM((KB,), jnp.int32),
            pltpu.VMEM((KB,), f32),
            pltpu.VMEM((KB, dc), f32),
            pltpu.VMEM((16,), jnp.int32),
            pltpu.VMEM((16,), jnp.int32),
            pltpu.VMEM((16,), f32),
            pltpu.VMEM((16, dc), f32),
            pltpu.VMEM((ZR, dc), f32),
            pltpu.SemaphoreType.DMA,
        ],
    )
    def k(tab_hbm, src_hbm, dst_hbm, ew_hbm, out_hbm,
          acc, sidx, didx, ewv, rows, sidx8, didx8, ewv8, rows8, zeros, sem):
        core = lax.axis_index("c")
        sub = lax.axis_index("s")
        w = sub * NC + core
        e0 = w * EPW
        r0 = pl.multiple_of(sub * CP, 8)

        def zrow(r, carry):
            for f in range(nv):
                zeros[r, pl.ds(f * 16, 16)] = jnp.zeros((16,), f32)
            return carry
        lax.fori_loop(0, ZR, zrow, 0)

        def scale_rows(ref, wref, nrows):
            def mul_body(g, carry):
                ew16 = wref[pl.ds(g * 16, 16)]
                for lane in range(16):
                    sv = ew16[lane]
                    rb = g * 16 + lane
                    for f in range(nv):
                        ref[rb, pl.ds(f * 16, 16)] = (
                            ref[rb, pl.ds(f * 16, 16)] * sv)
                return carry
            lax.fori_loop(0, nrows // 16, mul_body, 0)

        for chunk in range(nchunk):
            for r in range(CP // ZR):
                pltpu.sync_copy(zeros, acc.at[pl.ds(r0 + r * ZR, ZR)])

            @pl.when(sub == 0)
            def _():
                pltpu.sync_copy(zeros.at[pl.ds(0, REM)],
                                acc.at[pl.ds(NS * CP, REM)])
            plsc.subcore_barrier()

            def batch_body(b, carry):
                base = e0 + b * KB
                pltpu.sync_copy(src_hbm.at[pl.ds(base, KB)], sidx)
                if chunk > 0:
                    for j in range(KB // 16):
                        sidx[pl.ds(j * 16, 16)] = (
                            sidx[pl.ds(j * 16, 16)] + chunk * N)
                pltpu.async_copy(tab_hbm.at[sidx], rows, sem).wait()
                pltpu.sync_copy(dst_hbm.at[pl.ds(base, KB)], didx)
                pltpu.sync_copy(ew_hbm.at[pl.ds(base, KB)], ewv)
                scale_rows(rows, ewv, KB)
                pltpu.sync_copy(rows, acc.at[didx], add=True)
                return carry
            lax.fori_loop(0, NB_FULL, batch_body, 0)

            if TAIL:
                base = e0 + NB_FULL * KB
                pltpu.sync_copy(src_hbm.at[pl.ds(base, TAIL)],
                                sidx8.at[pl.ds(0, TAIL)])
                pltpu.sync_copy(dst_hbm.at[pl.ds(base, TAIL)],
                                didx8.at[pl.ds(0, TAIL)])
                pltpu.sync_copy(ew_hbm.at[pl.ds(base, TAIL)],
                                ewv8.at[pl.ds(0, TAIL)])
                lane = lax.iota(jnp.int32, 16)
                m = lane < TAIL
                sidx8[...] = jnp.where(m, sidx8[...] + chunk * N, 0)
                didx8[...] = jnp.where(m, didx8[...], 0)
                ewv8[...] = jnp.where(m, ewv8[...], 0.0)
                pltpu.async_copy(tab_hbm.at[sidx8], rows8, sem).wait()
                scale_rows(rows8, ewv8, 16)
                pltpu.sync_copy(rows8, acc.at[didx8], add=True)

            plsc.subcore_barrier()
            for r in range(CP // ZR):
                sl = pl.ds(r0 + r * ZR, ZR)
                pltpu.sync_copy(acc.at[sl], out_hbm.at[core, chunk, sl])

            @pl.when(sub == 0)
            def _():
                sl = pl.ds(NS * CP, REM)
                pltpu.sync_copy(acc.at[sl], out_hbm.at[core, chunk, sl])
            plsc.subcore_barrier()

    return k


GCN = [5, 32, 128, 256, 512, 512, 256, 256]
MLPDIMS = [256, 256, 128, 64]


def kernel(x, edge_index, edge_attr, batch, bit_width,
           Wrel, brel, Wroot, Wd, bd, Wout, bout):
    ea2d = edge_attr.reshape(1250, 128)
    qm = (jnp.float32(2.0) ** bit_width - 1.0).reshape(1, 1)
    src = edge_index[0]
    dst = edge_index[1]
    lo_x, hi_x = _minmax(x)
    lo_e, hi_e = _minmax(ea2d)

    for i in range(7):
        fin, fout = GCN[i], GCN[i + 1]
        pre = i >= 4
        d = fout if pre else fin
        dpad = max(16, d)
        dc = min(dpad, 128)
        nchunk = dpad // dc
        lo = jnp.minimum(lo_x, lo_e)
        hi = jnp.maximum(hi_x, hi_e)
        tab, root = _quantmm(x, lo, hi, qm, Wrel[i], Wroot[i],
                             brel[i].reshape(1, fout), pre, nchunk, dc)
        ea2d, lo_e, hi_e = _eaquant(ea2d, lo, hi, qm)
        parts = _make_scatter(nchunk, dc)(
            tab.reshape(nchunk * N, dc), src, dst, ea2d.reshape(E))
        x, lo_x, hi_x = _combine(parts, root, None if pre else Wrel[i],
                                 fin, nchunk, dc)

    sums, counts = _pool(x, batch.reshape(N, 1))
    out = _mlp(sums, counts, lo_e, hi_e, qm,
               [w for w in Wd], [b.reshape(1, -1) for b in bd],
               Wout, bout.reshape(1, 1))
    return out
